# baseline (speedup 1.0000x reference)
import functools

import jax
import jax.numpy as jnp
import numpy as np
from jax import lax
from jax.experimental import pallas as pl
from jax.experimental.pallas import tpu as pltpu
from jax.experimental.pallas import tpu_sc as plsc

H = 4
S = 8
P = 65536
F = 16
V = 64 * 64 * 64
Q = F // 2

NC = 2
NS = 16
PT = P // NS
HEADS_PER_CORE = H // NC
PLANE_CHUNK = V // NS
UNROLL = 4
HALF = (S // 2) * PT

_HI_MASK = np.int32(np.uint32(0xFFFF0000).view(np.int32))


def _expand(word):
    lo = plsc.bitcast(word << 16, jnp.float32)
    hi = plsc.bitcast(word & _HI_MASK, jnp.float32)
    return lo, hi


def _sc_body(lc_hbm, fi_hbm, cv_hbm, out_hbm, plane_a, plane_b, w_v,
             out_v, idx_v, g_a, g_b, *sems):
    stage_sem, gather_sem, out_sem = sems
    planes = (plane_a, plane_b)
    g_half = (g_a, g_b)

    cid = lax.axis_index("c")
    sid = lax.axis_index("s")

    def stage_plane(prow, buf):
        return pltpu.async_copy(
            cv_hbm.at[prow, pl.ds(sid * PLANE_CHUNK, PLANE_CHUNK)],
            buf.at[pl.ds(sid * PLANE_CHUNK, PLANE_CHUNK)],
            stage_sem,
        )

    out_cps = []
    for h2 in range(HEADS_PER_CORE):
        h = cid * HEADS_PER_CORE + h2
        for s in range(S):
            pltpu.sync_copy(fi_hbm.at[h, s, sid, :], idx_v.at[pl.ds(s * PT, PT)])
        pltpu.sync_copy(lc_hbm.at[h, :, sid, :], w_v)

        cp = stage_plane(h * Q, planes[0])
        for q in range(Q):
            prow = h * Q + q
            pb = q % 2
            cp.wait()
            plsc.subcore_barrier()
            if q + 1 < Q:
                cp = stage_plane(prow + 1, planes[1 - pb])

            gcp_a = pltpu.async_copy(
                planes[pb].at[idx_v.at[pl.ds(0, HALF)]], g_a, gather_sem)
            gcp_a.wait()
            gcp_b = pltpu.async_copy(
                planes[pb].at[idx_v.at[pl.ds(HALF, HALF)]], g_b, gather_sem)

            for ocp in out_cps:
                ocp.wait()
            out_cps = []

            for half in range(2):
                if half == 1:
                    gcp_b.wait()
                g_v = g_half[half]

                def _chunk_body(j, _, half=half, g_v=g_v):
                    base = j * (16 * UNROLL)
                    for u in range(UNROLL):
                        col = base + u * 16
                        acc0 = acc1 = None
                        for sp in range(2):
                            wword = w_v[2 * half + sp, pl.ds(col, 16)]
                            wa, wb = _expand(wword)
                            for k, wgt in ((0, wa), (1, wb)):
                                s_loc = 2 * sp + k
                                gw = g_v[pl.ds(s_loc * PT + col, 16)]
                                g0, g1 = _expand(gw)
                                t0 = wgt * g0
                                t1 = wgt * g1
                                acc0 = t0 if acc0 is None else acc0 + t0
                                acc1 = t1 if acc1 is None else acc1 + t1
                        if half == 0:
                            out_v[0, pl.ds(col, 16)] = acc0
                            out_v[1, pl.ds(col, 16)] = acc1
                        else:
                            out_v[0, pl.ds(col, 16)] += acc0
                            out_v[1, pl.ds(col, 16)] += acc1
                    return 0

                lax.fori_loop(0, PT // (16 * UNROLL), _chunk_body, 0)

            for k in range(2):
                out_cps.append(pltpu.async_copy(
                    out_v.at[k],
                    out_hbm.at[h * F + 2 * q + k, pl.ds(sid * PT, PT)],
                    out_sem))

    for ocp in out_cps:
        ocp.wait()


@jax.jit
def _slice_sc(lc, fi, cv):
    mesh = plsc.VectorSubcoreMesh(
        core_axis_name="c", subcore_axis_name="s", num_cores=NC, num_subcores=NS
    )
    run = pl.kernel(
        _sc_body,
        out_type=jax.ShapeDtypeStruct((H * F, P), jnp.float32),
        mesh=mesh,
        compiler_params=pltpu.CompilerParams(needs_layout_passes=False),
        scratch_types=[
            pltpu.VMEM_SHARED((V,), jnp.int32),
            pltpu.VMEM_SHARED((V,), jnp.int32),
            pltpu.VMEM((S // 2, PT), jnp.int32),
            pltpu.VMEM((2, PT), jnp.float32),
            pltpu.VMEM((S * PT,), jnp.int32),
            pltpu.VMEM((HALF,), jnp.int32),
            pltpu.VMEM((HALF,), jnp.int32),
        ]
        + [pltpu.SemaphoreType.DMA] * 3,
    )
    return run(lc, fi, cv)


def _rne_bf16_bits(x):
    b = jax.lax.bitcast_convert_type(x, jnp.uint32)
    lsb = (b >> 16) & jnp.uint32(1)
    return (b + jnp.uint32(0x7FFF) + lsb) >> 16


def _pack_pairs(lo_f32, hi_f32):
    word = _rne_bf16_bits(lo_f32) | (_rne_bf16_bits(hi_f32) << 16)
    return jax.lax.bitcast_convert_type(word, jnp.int32)


def kernel(local_coordinate, flattened_index, convolved):
    lcr = local_coordinate.reshape(H * S, P)
    lc = _pack_pairs(lcr[0::2], lcr[1::2]).reshape(H, S // 2, NS, PT)
    fi = flattened_index.reshape(H, S, NS, PT).astype(jnp.int32)
    cvr = convolved.reshape(H * F, V)
    cv = _pack_pairs(cvr[0::2], cvr[1::2])
    out = _slice_sc(lc, fi, cv)
    return out.reshape(1, H * F, P)


# speedup vs baseline: 1.8744x; 1.8744x over previous
import functools

import jax
import jax.numpy as jnp
import numpy as np
from jax import lax
from jax.experimental import pallas as pl
from jax.experimental.pallas import tpu as pltpu
from jax.experimental.pallas import tpu_sc as plsc

H = 4
S = 8
P = 65536
F = 16
V = 64 * 64 * 64
Q = F // 2

NC = 2
NS = 16
PT = P // NS
HEADS_PER_CORE = H // NC
PLANE_CHUNK = V // NS
UNROLL = 4
HALF = (S // 2) * PT

_HI_MASK = np.int32(np.uint32(0xFFFF0000).view(np.int32))


def _expand(word):
    lo = plsc.bitcast(word << 16, jnp.float32)
    hi = plsc.bitcast(word & _HI_MASK, jnp.float32)
    return lo, hi


def _sc_body(lc_hbm, fi_hbm, cv_hbm, out_hbm, plane_a, plane_b, w_v,
             out_v, idx_v, g_a, g_b, *sems):
    stage_sem, gather_sem, out_sem = sems
    planes = (plane_a, plane_b)
    g_half = (g_a, g_b)

    cid = lax.axis_index("c")
    sid = lax.axis_index("s")

    def stage_plane(prow, buf):
        return pltpu.async_copy(
            cv_hbm.at[prow, pl.ds(sid * PLANE_CHUNK, PLANE_CHUNK)],
            buf.at[pl.ds(sid * PLANE_CHUNK, PLANE_CHUNK)],
            stage_sem,
        )

    out_cps = []
    for h2 in range(HEADS_PER_CORE):
        h = cid * HEADS_PER_CORE + h2
        for s in range(S):
            pltpu.sync_copy(fi_hbm.at[h, s, sid, :], idx_v.at[pl.ds(s * PT, PT)])
        pltpu.sync_copy(lc_hbm.at[h, :, sid, :], w_v)

        cp = stage_plane(h * Q, planes[0])
        for q in range(Q):
            prow = h * Q + q
            pb = q % 2
            cp.wait()
            plsc.subcore_barrier()
            if q + 1 < Q:
                cp = stage_plane(prow + 1, planes[1 - pb])

            gcp_a = pltpu.async_copy(
                planes[pb].at[idx_v.at[pl.ds(0, HALF)]], g_a, gather_sem)
            gcp_a.wait()
            gcp_b = pltpu.async_copy(
                planes[pb].at[idx_v.at[pl.ds(HALF, HALF)]], g_b, gather_sem)

            for ocp in out_cps:
                ocp.wait()
            out_cps = []

            for half in range(2):
                if half == 1:
                    gcp_b.wait()
                g_v = g_half[half]

                def _chunk_body(j, _, half=half, g_v=g_v):
                    base = j * (16 * UNROLL)
                    for u in range(UNROLL):
                        col = base + u * 16
                        acc0 = acc1 = None
                        for sp in range(S // 2):
                            wword = w_v[sp, pl.ds(col, 16)]
                            if half == 0:
                                wgt = plsc.bitcast(wword << 16, jnp.float32)
                            else:
                                wgt = plsc.bitcast(wword & _HI_MASK, jnp.float32)
                            gw = g_v[pl.ds(sp * PT + col, 16)]
                            g0, g1 = _expand(gw)
                            t0 = wgt * g0
                            t1 = wgt * g1
                            acc0 = t0 if acc0 is None else acc0 + t0
                            acc1 = t1 if acc1 is None else acc1 + t1
                        if half == 0:
                            out_v[0, pl.ds(col, 16)] = acc0
                            out_v[1, pl.ds(col, 16)] = acc1
                        else:
                            out_v[0, pl.ds(col, 16)] += acc0
                            out_v[1, pl.ds(col, 16)] += acc1
                    return 0

                lax.fori_loop(0, PT // (16 * UNROLL), _chunk_body, 0)

            for k in range(2):
                out_cps.append(pltpu.async_copy(
                    out_v.at[k],
                    out_hbm.at[h * F + q + k * Q, pl.ds(sid * PT, PT)],
                    out_sem))

    for ocp in out_cps:
        ocp.wait()


@jax.jit
def _slice_sc(lc, fi, cv):
    mesh = plsc.VectorSubcoreMesh(
        core_axis_name="c", subcore_axis_name="s", num_cores=NC, num_subcores=NS
    )
    run = pl.kernel(
        _sc_body,
        out_type=jax.ShapeDtypeStruct((H * F, P), jnp.float32),
        mesh=mesh,
        compiler_params=pltpu.CompilerParams(needs_layout_passes=False),
        scratch_types=[
            pltpu.VMEM_SHARED((V,), jnp.int32),
            pltpu.VMEM_SHARED((V,), jnp.int32),
            pltpu.VMEM((S // 2, PT), jnp.int32),
            pltpu.VMEM((2, PT), jnp.float32),
            pltpu.VMEM((S * PT,), jnp.int32),
            pltpu.VMEM((HALF,), jnp.int32),
            pltpu.VMEM((HALF,), jnp.int32),
        ]
        + [pltpu.SemaphoreType.DMA] * 3,
    )
    return run(lc, fi, cv)


def _rne_bf16_bits(x):
    b = jax.lax.bitcast_convert_type(x, jnp.uint32)
    lsb = (b >> 16) & jnp.uint32(1)
    return (b + jnp.uint32(0x7FFF) + lsb) >> 16


def _pack_pairs(lo_f32, hi_f32):
    word = _rne_bf16_bits(lo_f32) | (_rne_bf16_bits(hi_f32) << 16)
    return jax.lax.bitcast_convert_type(word, jnp.int32)


def kernel(local_coordinate, flattened_index, convolved):
    lc3 = local_coordinate.reshape(H, 2, S // 2, P)
    lc = _pack_pairs(lc3[:, 0], lc3[:, 1]).reshape(H, S // 2, NS, PT)
    fi = flattened_index.reshape(H, S, NS, PT).astype(jnp.int32)
    cv3 = convolved.reshape(H, 2, Q, V)
    cv = _pack_pairs(cv3[:, 0], cv3[:, 1]).reshape(H * Q, V)
    out = _slice_sc(lc, fi, cv)
    return out.reshape(1, H * F, P)


# speedup vs baseline: 2.1314x; 1.1371x over previous
import functools

import jax
import jax.numpy as jnp
import numpy as np
from jax import lax
from jax.experimental import pallas as pl
from jax.experimental.pallas import tpu as pltpu
from jax.experimental.pallas import tpu_sc as plsc

H = 4
S = 8
P = 65536
F = 16
V = 64 * 64 * 64
Q = F // 2

NC = 2
NS = 16
PT = P // NS
HEADS_PER_CORE = H // NC
PLANE_CHUNK = V // NS
UNROLL = 4
HALF = (S // 2) * PT

_HI_MASK = np.int32(np.uint32(0xFFFF0000).view(np.int32))


def _expand(word):
    lo = plsc.bitcast(word << 16, jnp.float32)
    hi = plsc.bitcast(word & _HI_MASK, jnp.float32)
    return lo, hi


def _sc_body(lc_hbm, fi_hbm, cv_hbm, out_hbm, plane_a, plane_b, w_v,
             out_v, idx_v, g_a, g_b, *sems):
    stage_sem, gather_sem, out_sem = sems
    planes = (plane_a, plane_b)
    g_half = (g_a, g_b)

    cid = lax.axis_index("c")
    sid = lax.axis_index("s")

    def stage_plane(prow, buf):
        return pltpu.async_copy(
            cv_hbm.at[prow, pl.ds(sid * PLANE_CHUNK, PLANE_CHUNK)],
            buf.at[pl.ds(sid * PLANE_CHUNK, PLANE_CHUNK)],
            stage_sem,
        )

    out_cps = []
    for h2 in range(HEADS_PER_CORE):
        h = cid * HEADS_PER_CORE + h2
        for s in range(S):
            pltpu.sync_copy(fi_hbm.at[h, s, sid, :], idx_v.at[pl.ds(s * PT, PT)])
        pltpu.sync_copy(lc_hbm.at[h, :, sid, :], w_v)

        cp = stage_plane(h * Q, planes[0])
        cp.wait()
        plsc.subcore_barrier()
        cp = stage_plane(h * Q + 1, planes[1])
        gcp_a = pltpu.async_copy(
            planes[0].at[idx_v.at[pl.ds(0, HALF)]], g_a, gather_sem)

        for q in range(Q):
            pb = q % 2
            gcp_a.wait()
            gcp_b = pltpu.async_copy(
                planes[pb].at[idx_v.at[pl.ds(HALF, HALF)]], g_b, gather_sem)

            for ocp in out_cps:
                ocp.wait()
            out_cps = []

            for half in range(2):
                if half == 1:
                    gcp_b.wait()
                    if q + 1 < Q:
                        cp.wait()
                        plsc.subcore_barrier()
                        if q + 2 < Q:
                            cp = stage_plane(h * Q + q + 2, planes[pb])
                        gcp_a = pltpu.async_copy(
                            planes[1 - pb].at[idx_v.at[pl.ds(0, HALF)]],
                            g_a, gather_sem)
                g_v = g_half[half]

                def _chunk_body(j, _, half=half, g_v=g_v):
                    base = j * (16 * UNROLL)
                    for u in range(UNROLL):
                        col = base + u * 16
                        acc0 = acc1 = None
                        for sp in range(S // 2):
                            wword = w_v[sp, pl.ds(col, 16)]
                            if half == 0:
                                wgt = plsc.bitcast(wword << 16, jnp.float32)
                            else:
                                wgt = plsc.bitcast(wword & _HI_MASK, jnp.float32)
                            gw = g_v[pl.ds(sp * PT + col, 16)]
                            g0, g1 = _expand(gw)
                            t0 = wgt * g0
                            t1 = wgt * g1
                            acc0 = t0 if acc0 is None else acc0 + t0
                            acc1 = t1 if acc1 is None else acc1 + t1
                        if half == 0:
                            out_v[0, pl.ds(col, 16)] = acc0
                            out_v[1, pl.ds(col, 16)] = acc1
                        else:
                            out_v[0, pl.ds(col, 16)] += acc0
                            out_v[1, pl.ds(col, 16)] += acc1
                    return 0

                lax.fori_loop(0, PT // (16 * UNROLL), _chunk_body, 0)

            for k in range(2):
                out_cps.append(pltpu.async_copy(
                    out_v.at[k],
                    out_hbm.at[h * F + q + k * Q, pl.ds(sid * PT, PT)],
                    out_sem))

    for ocp in out_cps:
        ocp.wait()


@jax.jit
def _slice_sc(lc, fi, cv):
    mesh = plsc.VectorSubcoreMesh(
        core_axis_name="c", subcore_axis_name="s", num_cores=NC, num_subcores=NS
    )
    run = pl.kernel(
        _sc_body,
        out_type=jax.ShapeDtypeStruct((H * F, P), jnp.float32),
        mesh=mesh,
        compiler_params=pltpu.CompilerParams(needs_layout_passes=False),
        scratch_types=[
            pltpu.VMEM_SHARED((V,), jnp.int32),
            pltpu.VMEM_SHARED((V,), jnp.int32),
            pltpu.VMEM((S // 2, PT), jnp.int32),
            pltpu.VMEM((2, PT), jnp.float32),
            pltpu.VMEM((S * PT,), jnp.int32),
            pltpu.VMEM((HALF,), jnp.int32),
            pltpu.VMEM((HALF,), jnp.int32),
        ]
        + [pltpu.SemaphoreType.DMA] * 3,
    )
    return run(lc, fi, cv)


def _rne_bf16_bits(x):
    b = jax.lax.bitcast_convert_type(x, jnp.uint32)
    lsb = (b >> 16) & jnp.uint32(1)
    return (b + jnp.uint32(0x7FFF) + lsb) >> 16


def _pack_pairs(lo_f32, hi_f32):
    word = _rne_bf16_bits(lo_f32) | (_rne_bf16_bits(hi_f32) << 16)
    return jax.lax.bitcast_convert_type(word, jnp.int32)


def kernel(local_coordinate, flattened_index, convolved):
    lc3 = local_coordinate.reshape(H, 2, S // 2, P)
    lc = _pack_pairs(lc3[:, 0], lc3[:, 1]).reshape(H, S // 2, NS, PT)
    fi = flattened_index.reshape(H, S, NS, PT).astype(jnp.int32)
    cv3 = convolved.reshape(H, 2, Q, V)
    cv = _pack_pairs(cv3[:, 0], cv3[:, 1]).reshape(H * Q, V)
    out = _slice_sc(lc, fi, cv)
    return out.reshape(1, H * F, P)
